# baseline (device time: 25340 ns/iter reference)
import jax
import jax.numpy as jnp
from jax import lax
from jax.experimental import pallas as pl
from jax.experimental.pallas import tpu as pltpu

K = 16


def _topk_desc(xv, k):
    neg_inf = jnp.float32(-jnp.inf)
    vals = []
    for i in range(k):
        mx = jnp.max(xv, axis=1, keepdims=True)
        vals.append(mx)
        if i < k - 1:
            xv = jnp.where(xv == mx, neg_inf, xv)
    return jnp.concatenate(vals, axis=1)


def kernel(x):
    m, n = x.shape

    def body(x_ref, out_ref, comm_ref, send_sem, recv_sem):
        my_x = lax.axis_index("x")
        my_y = lax.axis_index("y")
        nbr = (my_x, 1 - my_y)

        barrier_sem = pltpu.get_barrier_semaphore()
        pl.semaphore_signal(
            barrier_sem, inc=1, device_id=nbr,
            device_id_type=pl.DeviceIdType.MESH,
        )

        local = _topk_desc(x_ref[:, :], K)
        comm_ref[0, :, :] = local

        pl.semaphore_wait(barrier_sem, 1)

        rdma = pltpu.make_async_remote_copy(
            src_ref=comm_ref.at[0],
            dst_ref=comm_ref.at[1],
            send_sem=send_sem,
            recv_sem=recv_sem,
            device_id=nbr,
            device_id_type=pl.DeviceIdType.MESH,
        )
        rdma.start()
        rdma.wait()

        both = jnp.concatenate([local, comm_ref[1, :, :]], axis=1)
        out_ref[:, :] = _topk_desc(both, K)

    return pl.pallas_call(
        body,
        out_shape=jax.ShapeDtypeStruct((m, K), jnp.float32),
        in_specs=[pl.BlockSpec(memory_space=pltpu.VMEM)],
        out_specs=pl.BlockSpec(memory_space=pltpu.VMEM),
        scratch_shapes=[
            pltpu.VMEM((2, m, K), jnp.float32),
            pltpu.SemaphoreType.DMA,
            pltpu.SemaphoreType.DMA,
        ],
        compiler_params=pltpu.CompilerParams(collective_id=0),
    )(x)


# device time: 25009 ns/iter; 1.0132x vs baseline; 1.0132x over previous
import jax
import jax.numpy as jnp
from jax import lax
from jax.experimental import pallas as pl
from jax.experimental.pallas import tpu as pltpu

K = 16


def _topk_desc(xv, k):
    neg_inf = jnp.float32(-jnp.inf)
    t = jnp.max(xv, axis=1, keepdims=True)
    vals = [t]
    for _ in range(k - 1):
        t = jnp.max(jnp.where(xv < t, xv, neg_inf), axis=1, keepdims=True)
        vals.append(t)
    return jnp.concatenate(vals, axis=1)


def kernel(x):
    m, n = x.shape

    def body(x_ref, out_ref, comm_ref, send_sem, recv_sem):
        my_x = lax.axis_index("x")
        my_y = lax.axis_index("y")
        nbr = (my_x, 1 - my_y)

        barrier_sem = pltpu.get_barrier_semaphore()
        pl.semaphore_signal(
            barrier_sem, inc=1, device_id=nbr,
            device_id_type=pl.DeviceIdType.MESH,
        )

        local = _topk_desc(x_ref[:, :], K)
        comm_ref[0, :, :] = local

        pl.semaphore_wait(barrier_sem, 1)

        rdma = pltpu.make_async_remote_copy(
            src_ref=comm_ref.at[0],
            dst_ref=comm_ref.at[1],
            send_sem=send_sem,
            recv_sem=recv_sem,
            device_id=nbr,
            device_id_type=pl.DeviceIdType.MESH,
        )
        rdma.start()
        rdma.wait()

        both = jnp.concatenate([local, comm_ref[1, :, :]], axis=1)
        out_ref[:, :] = _topk_desc(both, K)

    return pl.pallas_call(
        body,
        out_shape=jax.ShapeDtypeStruct((m, K), jnp.float32),
        in_specs=[pl.BlockSpec(memory_space=pltpu.VMEM)],
        out_specs=pl.BlockSpec(memory_space=pltpu.VMEM),
        scratch_shapes=[
            pltpu.VMEM((2, m, K), jnp.float32),
            pltpu.SemaphoreType.DMA,
            pltpu.SemaphoreType.DMA,
        ],
        compiler_params=pltpu.CompilerParams(collective_id=0),
    )(x)


# device time: 20470 ns/iter; 1.2379x vs baseline; 1.2217x over previous
import jax
import jax.numpy as jnp
from jax import lax
from jax.experimental import pallas as pl
from jax.experimental.pallas import tpu as pltpu

K = 16


def _topk_desc(xv, k):
    neg_inf = jnp.float32(-jnp.inf)
    t = jnp.max(xv, axis=1, keepdims=True)
    vals = [t]
    for _ in range(k - 1):
        t = jnp.max(jnp.where(xv < t, xv, neg_inf), axis=1, keepdims=True)
        vals.append(t)
    return jnp.concatenate(vals, axis=1)


_LEAF_WIDTH = 512


def _topk_candidates(xv, k):
    w = xv.shape[1]
    if k == 1:
        return jnp.max(xv, axis=1, keepdims=True)
    if w <= _LEAF_WIDTH:
        return _topk_desc(xv, k)
    half = w // 2
    left, right = xv[:, :half], xv[:, half:]
    hi = jnp.maximum(left, right)
    lo = jnp.minimum(left, right)
    return jnp.concatenate(
        [_topk_candidates(hi, k), _topk_candidates(lo, k // 2)], axis=1
    )


def kernel(x):
    m, n = x.shape

    def body(x_ref, out_ref, comm_ref, send_sem, recv_sem):
        my_x = lax.axis_index("x")
        my_y = lax.axis_index("y")
        nbr = (my_x, 1 - my_y)

        barrier_sem = pltpu.get_barrier_semaphore()
        pl.semaphore_signal(
            barrier_sem, inc=1, device_id=nbr,
            device_id_type=pl.DeviceIdType.MESH,
        )

        local = _topk_desc(_topk_candidates(x_ref[:, :], K), K)
        comm_ref[0, :, :] = local

        pl.semaphore_wait(barrier_sem, 1)

        rdma = pltpu.make_async_remote_copy(
            src_ref=comm_ref.at[0],
            dst_ref=comm_ref.at[1],
            send_sem=send_sem,
            recv_sem=recv_sem,
            device_id=nbr,
            device_id_type=pl.DeviceIdType.MESH,
        )
        rdma.start()
        rdma.wait()

        both = jnp.concatenate([local, comm_ref[1, :, :]], axis=1)
        out_ref[:, :] = _topk_desc(both, K)

    return pl.pallas_call(
        body,
        out_shape=jax.ShapeDtypeStruct((m, K), jnp.float32),
        in_specs=[pl.BlockSpec(memory_space=pltpu.VMEM)],
        out_specs=pl.BlockSpec(memory_space=pltpu.VMEM),
        scratch_shapes=[
            pltpu.VMEM((2, m, K), jnp.float32),
            pltpu.SemaphoreType.DMA,
            pltpu.SemaphoreType.DMA,
        ],
        compiler_params=pltpu.CompilerParams(collective_id=0),
    )(x)
